# Initial kernel scaffold; baseline (speedup 1.0000x reference)
import jax
import jax.numpy as jnp
from jax.experimental import pallas as pl


def kernel(static, known_real, known_categorical, observed, static_emb, known_cat_emb, known_real_W, known_real_b, observed_W, observed_b):
    raise NotImplementedError("write your pallas kernel here")


# speedup vs baseline: 4.3998x; 4.3998x over previous
import functools

import jax
import jax.numpy as jnp
from jax import lax
from jax.experimental import pallas as pl
from jax.experimental.pallas import tpu as pltpu
from jax.experimental.pallas import tpu_sc as plsc

H = 32
B = 1024
T = 200
N_STATIC = 4
STATIC_VOCAB = 100000
N_KNOWN_CAT = 3
KNOWN_VOCAB = 1000
VPAD = 1024

NC = 2
NS = 16
NW = NC * NS

UNITS = T * (B // 128)
UPW = UNITS // NW
TABF = N_KNOWN_CAT * H * VPAD

STATIC_ROWS = B * N_STATIC
SPW = STATIC_ROWS // NW


@functools.partial(
    pl.kernel,
    mesh=plsc.VectorSubcoreMesh(core_axis_name="c", subcore_axis_name="s"),
    out_type=jax.ShapeDtypeStruct((T, 6 * H, B), jnp.float32),
    scratch_types=[
        pltpu.VMEM((TABF,), jnp.float32),
        pltpu.VMEM((N_KNOWN_CAT, 128), jnp.int32),
        pltpu.VMEM((N_KNOWN_CAT * H, 128), jnp.float32),
    ],
    compiler_params=pltpu.CompilerParams(needs_layout_passes=False),
)
def _sc_cat_gather(tab1d, idxu, out, tab_v, idx_v, buf_v):
    wid = lax.axis_index("s") * NC + lax.axis_index("c")
    pltpu.sync_copy(tab1d, tab_v)

    def unit(u, carry):
        uid = wid * UPW + u
        t = uid // 8
        b0 = pl.multiple_of((uid % 8) * 128, 128)
        pltpu.sync_copy(idxu.at[uid], idx_v)
        for c in range(N_KNOWN_CAT):
            iv = [idx_v[c, pl.ds(16 * bb, 16)] for bb in range(8)]
            for h in range(H):
                base = (c * H + h) * VPAD
                for bb in range(8):
                    g = plsc.load_gather(tab_v, [iv[bb] + base])
                    buf_v[c * H + h, pl.ds(16 * bb, 16)] = g
        pltpu.sync_copy(buf_v, out.at[t, pl.ds(96, 96), pl.ds(b0, 128)])
        return carry

    lax.fori_loop(0, UPW, unit, 0)


@functools.partial(
    pl.kernel,
    mesh=plsc.VectorSubcoreMesh(core_axis_name="c", subcore_axis_name="s"),
    out_type=jax.ShapeDtypeStruct((NW, SPW, H), jnp.float32),
    scratch_types=[
        pltpu.VMEM((1, SPW), jnp.int32),
        pltpu.VMEM((SPW, H), jnp.float32),
        pltpu.SemaphoreType.DMA,
    ],
    compiler_params=pltpu.CompilerParams(use_tc_tiling_on_sc=False),
)
def _sc_static_gather(tstat, idxs, out, sidx_v, srows_v, sem):
    wid = lax.axis_index("s") * NC + lax.axis_index("c")
    pltpu.sync_copy(idxs.at[wid], sidx_v)
    pltpu.async_copy(tstat.at[sidx_v.at[0]], srows_v, sem).wait()
    pltpu.sync_copy(srows_v, out.at[wid])


def _tc_body(kpre_ref, knr_ref, obs_ref, mk_ref, mo_ref, known_ref, obs_out_ref):
    del kpre_ref
    known_ref[0] = jax.lax.dot(mk_ref[...], knr_ref[0], precision="highest",
                               preferred_element_type=jnp.float32)
    obs_out_ref[0] = jax.lax.dot(mo_ref[...], obs_ref[0], precision="highest",
                                 preferred_element_type=jnp.float32)


def _tc_combine(kpre, knr_aug, obs_aug, mk, mo):
    return pl.pallas_call(
        _tc_body,
        grid=(T,),
        in_specs=[
            pl.BlockSpec(memory_space=pl.ANY),
            pl.BlockSpec((1, 4, B), lambda t: (t, 0, 0)),
            pl.BlockSpec((1, 4, B), lambda t: (t, 0, 0)),
            pl.BlockSpec((3 * H, 4), lambda t: (0, 0)),
            pl.BlockSpec((3 * H, 4), lambda t: (0, 0)),
        ],
        out_specs=[
            pl.BlockSpec((1, 3 * H, B), lambda t: (t, 0, 0)),
            pl.BlockSpec((1, 3 * H, B), lambda t: (t, 0, 0)),
        ],
        out_shape=[
            jax.ShapeDtypeStruct((T, 6 * H, B), jnp.float32),
            jax.ShapeDtypeStruct((T, 3 * H, B), jnp.float32),
        ],
        input_output_aliases={0: 0},
    )(kpre, knr_aug, obs_aug, mk, mo)


def kernel(static, known_real, known_categorical, observed,
           static_emb, known_cat_emb,
           known_real_W, known_real_b, observed_W, observed_b):
    f32 = jnp.float32

    tabT = jnp.transpose(known_cat_emb, (0, 2, 1))
    tab1d = jnp.pad(tabT, ((0, 0), (0, 0), (0, VPAD - KNOWN_VOCAB))).reshape(-1)
    idxT = jnp.transpose(known_categorical, (1, 2, 0))
    idxu = (idxT.reshape(T, N_KNOWN_CAT, 8, 128)
            .transpose(0, 2, 1, 3).reshape(UNITS, N_KNOWN_CAT, 128))
    kpre = _sc_cat_gather(tab1d, idxu)

    tstat = static_emb.reshape(N_STATIC * STATIC_VOCAB, H)
    idx_stat = (static + STATIC_VOCAB * jnp.arange(N_STATIC, dtype=jnp.int32)
                ).reshape(NW, 1, SPW)
    static_rows = _sc_static_gather(tstat, idx_stat)
    static_out = static_rows.reshape(B, N_STATIC, H)

    rr = jnp.arange(3 * H)
    mk = (jnp.zeros((3 * H, 4), f32)
          .at[rr, rr // H].set(known_real_W.reshape(-1))
          .at[:, 3].set(known_real_b.reshape(-1)))
    mo = (jnp.zeros((3 * H, 4), f32)
          .at[rr, rr // H].set(observed_W.reshape(-1))
          .at[:, 3].set(observed_b.reshape(-1)))
    ones_row = jnp.ones((T, 1, B), f32)
    knr_aug = jnp.concatenate([jnp.transpose(known_real, (1, 2, 0)), ones_row], axis=1)
    obs_aug = jnp.concatenate([jnp.transpose(observed, (1, 2, 0)), ones_row], axis=1)

    known_pre, obs_pre = _tc_combine(kpre, knr_aug, obs_aug, mk, mo)

    known_out = jnp.transpose(known_pre.reshape(T, 6, H, B), (3, 0, 2, 1))
    observed_out = jnp.transpose(obs_pre.reshape(T, 3, H, B), (3, 0, 2, 1))
    return static_out, known_out, observed_out


# speedup vs baseline: 4.6527x; 1.0575x over previous
import functools

import jax
import jax.numpy as jnp
from jax import lax
from jax.experimental import pallas as pl
from jax.experimental.pallas import tpu as pltpu
from jax.experimental.pallas import tpu_sc as plsc

H = 32
B = 1024
T = 200
N_STATIC = 4
STATIC_VOCAB = 100000
N_KNOWN_CAT = 3
KNOWN_VOCAB = 1000
VPAD = 1024

NC = 2
NS = 16
NW = NC * NS

UNITS = T * (B // 128)
UPW = UNITS // NW
TABF = N_KNOWN_CAT * H * VPAD

STATIC_ROWS = B * N_STATIC
SPW = STATIC_ROWS // NW


@functools.partial(
    pl.kernel,
    mesh=plsc.VectorSubcoreMesh(core_axis_name="c", subcore_axis_name="s"),
    out_type=jax.ShapeDtypeStruct((T, 6 * H, B), jnp.float32),
    scratch_types=[
        pltpu.VMEM((TABF,), jnp.float32),
        pltpu.VMEM((N_KNOWN_CAT, 128), jnp.int32),
        pltpu.VMEM((N_KNOWN_CAT, 128), jnp.int32),
        pltpu.VMEM((N_KNOWN_CAT * H, 128), jnp.float32),
        pltpu.VMEM((N_KNOWN_CAT * H, 128), jnp.float32),
        pltpu.SemaphoreType.DMA,
        pltpu.SemaphoreType.DMA,
    ],
    compiler_params=pltpu.CompilerParams(needs_layout_passes=False),
)
def _sc_cat_gather(tab1d, idxu, out, tab_v, idx_v0, idx_v1, buf_v0, buf_v1,
                   sem_in, sem_out):
    wid = lax.axis_index("s") * NC + lax.axis_index("c")
    pltpu.sync_copy(tab1d, tab_v)
    idx_bufs = (idx_v0, idx_v1)
    bufs = (buf_v0, buf_v1)

    pltpu.async_copy(idxu.at[wid * UPW], idx_v0, sem_in)

    def pair(p, carry):
        for k in range(2):
            u = 2 * p + k
            uid = wid * UPW + u
            t = uid // 8
            b0 = pl.multiple_of((uid % 8) * 128, 128)
            icur, bcur = idx_bufs[k], bufs[k]
            pltpu.make_async_copy(idxu.at[uid], icur, sem_in).wait()
            nxt = lax.min(uid + 1, UNITS - 1)
            pltpu.async_copy(idxu.at[nxt], idx_bufs[1 - k], sem_in)

            @pl.when(u >= 2)
            def _():
                pltpu.make_async_copy(
                    bcur, out.at[0, pl.ds(96, 96), pl.ds(0, 128)], sem_out
                ).wait()

            for c in range(N_KNOWN_CAT):
                iv = [icur[c, pl.ds(16 * bb, 16)] for bb in range(8)]
                for h in range(H):
                    base = (c * H + h) * VPAD
                    for bb in range(8):
                        g = plsc.load_gather(tab_v, [iv[bb] + base])
                        bcur[c * H + h, pl.ds(16 * bb, 16)] = g
            pltpu.async_copy(bcur, out.at[t, pl.ds(96, 96), pl.ds(b0, 128)],
                             sem_out)
        return carry

    lax.fori_loop(0, UPW // 2, pair, 0)
    pltpu.make_async_copy(idxu.at[0], idx_v0, sem_in).wait()
    pltpu.make_async_copy(buf_v0, out.at[0, pl.ds(96, 96), pl.ds(0, 128)],
                          sem_out).wait()
    pltpu.make_async_copy(buf_v1, out.at[0, pl.ds(96, 96), pl.ds(0, 128)],
                          sem_out).wait()


@functools.partial(
    pl.kernel,
    mesh=plsc.VectorSubcoreMesh(core_axis_name="c", subcore_axis_name="s"),
    out_type=jax.ShapeDtypeStruct((NW, SPW, H), jnp.float32),
    scratch_types=[
        pltpu.VMEM((1, SPW), jnp.int32),
        pltpu.VMEM((SPW, H), jnp.float32),
        pltpu.SemaphoreType.DMA,
    ],
    compiler_params=pltpu.CompilerParams(use_tc_tiling_on_sc=False),
)
def _sc_static_gather(tstat, idxs, out, sidx_v, srows_v, sem):
    wid = lax.axis_index("s") * NC + lax.axis_index("c")
    pltpu.sync_copy(idxs.at[wid], sidx_v)
    pltpu.async_copy(tstat.at[sidx_v.at[0]], srows_v, sem).wait()
    pltpu.sync_copy(srows_v, out.at[wid])


def _tc_body(kpre_ref, knr_ref, obs_ref, mk_ref, mo_ref, known_ref, obs_out_ref):
    del kpre_ref
    known_ref[0] = jax.lax.dot(mk_ref[...], knr_ref[0], precision="highest",
                               preferred_element_type=jnp.float32)
    obs_out_ref[0] = jax.lax.dot(mo_ref[...], obs_ref[0], precision="highest",
                                 preferred_element_type=jnp.float32)


def _tc_combine(kpre, knr_aug, obs_aug, mk, mo):
    return pl.pallas_call(
        _tc_body,
        grid=(T,),
        in_specs=[
            pl.BlockSpec(memory_space=pl.ANY),
            pl.BlockSpec((1, 4, B), lambda t: (t, 0, 0)),
            pl.BlockSpec((1, 4, B), lambda t: (t, 0, 0)),
            pl.BlockSpec((3 * H, 4), lambda t: (0, 0)),
            pl.BlockSpec((3 * H, 4), lambda t: (0, 0)),
        ],
        out_specs=[
            pl.BlockSpec((1, 3 * H, B), lambda t: (t, 0, 0)),
            pl.BlockSpec((1, 3 * H, B), lambda t: (t, 0, 0)),
        ],
        out_shape=[
            jax.ShapeDtypeStruct((T, 6 * H, B), jnp.float32),
            jax.ShapeDtypeStruct((T, 3 * H, B), jnp.float32),
        ],
        input_output_aliases={0: 0},
    )(kpre, knr_aug, obs_aug, mk, mo)


def kernel(static, known_real, known_categorical, observed,
           static_emb, known_cat_emb,
           known_real_W, known_real_b, observed_W, observed_b):
    f32 = jnp.float32

    tabT = jnp.transpose(known_cat_emb, (0, 2, 1))
    tab1d = jnp.pad(tabT, ((0, 0), (0, 0), (0, VPAD - KNOWN_VOCAB))).reshape(-1)
    idxT = jnp.transpose(known_categorical, (1, 2, 0))
    idxu = (idxT.reshape(T, N_KNOWN_CAT, 8, 128)
            .transpose(0, 2, 1, 3).reshape(UNITS, N_KNOWN_CAT, 128))
    kpre = _sc_cat_gather(tab1d, idxu)

    tstat = static_emb.reshape(N_STATIC * STATIC_VOCAB, H)
    idx_stat = (static + STATIC_VOCAB * jnp.arange(N_STATIC, dtype=jnp.int32)
                ).reshape(NW, 1, SPW)
    static_rows = _sc_static_gather(tstat, idx_stat)
    static_out = static_rows.reshape(B, N_STATIC, H)

    rr = jnp.arange(3 * H)
    mk = (jnp.zeros((3 * H, 4), f32)
          .at[rr, rr // H].set(known_real_W.reshape(-1))
          .at[:, 3].set(known_real_b.reshape(-1)))
    mo = (jnp.zeros((3 * H, 4), f32)
          .at[rr, rr // H].set(observed_W.reshape(-1))
          .at[:, 3].set(observed_b.reshape(-1)))
    ones_row = jnp.ones((T, 1, B), f32)
    knr_aug = jnp.concatenate([jnp.transpose(known_real, (1, 2, 0)), ones_row], axis=1)
    obs_aug = jnp.concatenate([jnp.transpose(observed, (1, 2, 0)), ones_row], axis=1)

    known_pre, obs_pre = _tc_combine(kpre, knr_aug, obs_aug, mk, mo)

    known_out = jnp.transpose(known_pre.reshape(T, 6, H, B), (3, 0, 2, 1))
    observed_out = jnp.transpose(obs_pre.reshape(T, 3, H, B), (3, 0, 2, 1))
    return static_out, known_out, observed_out


# speedup vs baseline: 5.7356x; 1.2327x over previous
import functools

import jax
import jax.numpy as jnp
from jax import lax
from jax.experimental import pallas as pl
from jax.experimental.pallas import tpu as pltpu
from jax.experimental.pallas import tpu_sc as plsc

H = 32
B = 1024
T = 200
N_STATIC = 4
STATIC_VOCAB = 100000
N_KNOWN_CAT = 3
KNOWN_VOCAB = 1000
VPAD = 1024

NC = 2
NS = 16
NW = NC * NS

UNITS = T * (B // 128)
UPW = UNITS // NW
TABF = N_KNOWN_CAT * H * VPAD

STATIC_ROWS = B * N_STATIC
SPW = STATIC_ROWS // NW


@functools.partial(
    pl.kernel,
    mesh=plsc.VectorSubcoreMesh(core_axis_name="c", subcore_axis_name="s"),
    out_type=jax.ShapeDtypeStruct((T, 6 * H, B), jnp.float32),
    scratch_types=[
        pltpu.VMEM((TABF,), jnp.float32),
        pltpu.VMEM((N_KNOWN_CAT, 128), jnp.int32),
        pltpu.VMEM((N_KNOWN_CAT, 128), jnp.int32),
        pltpu.VMEM((N_KNOWN_CAT * H, 128), jnp.float32),
        pltpu.VMEM((N_KNOWN_CAT * H, 128), jnp.float32),
        pltpu.SemaphoreType.DMA,
        pltpu.SemaphoreType.DMA,
    ],
    compiler_params=pltpu.CompilerParams(needs_layout_passes=False),
)
def _sc_cat_gather(tab1d, idxu, out, tab_v, idx_v0, idx_v1, buf_v0, buf_v1,
                   sem_in, sem_out):
    wid = lax.axis_index("s") * NC + lax.axis_index("c")
    pltpu.sync_copy(tab1d, tab_v)
    idx_bufs = (idx_v0, idx_v1)
    bufs = (buf_v0, buf_v1)

    pltpu.async_copy(idxu.at[wid * UPW], idx_v0, sem_in)

    def pair(p, carry):
        for k in range(2):
            u = 2 * p + k
            uid = wid * UPW + u
            t = uid // 8
            b0 = pl.multiple_of((uid % 8) * 128, 128)
            icur, bcur = idx_bufs[k], bufs[k]
            pltpu.make_async_copy(idxu.at[uid], icur, sem_in).wait()
            nxt = lax.min(uid + 1, UNITS - 1)
            pltpu.async_copy(idxu.at[nxt], idx_bufs[1 - k], sem_in)

            @pl.when(u >= 2)
            def _():
                pltpu.make_async_copy(
                    bcur, out.at[0, pl.ds(96, 96), pl.ds(0, 128)], sem_out
                ).wait()

            iv = [[icur[c, pl.ds(16 * bb, 16)] for bb in range(8)]
                  for c in range(N_KNOWN_CAT)]
            prev, prev_row = None, 0
            for c in range(N_KNOWN_CAT):
                for h in range(H):
                    row = tab_v.at[pl.ds((c * H + h) * VPAD, VPAD)]
                    cur = []
                    for bb in range(8):
                        cur.append(plsc.load_gather(row, [iv[c][bb]]))
                        if prev is not None:
                            bcur[prev_row, pl.ds(16 * bb, 16)] = prev[bb]
                    prev, prev_row = cur, c * H + h
            for bb in range(8):
                bcur[prev_row, pl.ds(16 * bb, 16)] = prev[bb]
            pltpu.async_copy(bcur, out.at[t, pl.ds(96, 96), pl.ds(b0, 128)],
                             sem_out)
        return carry

    lax.fori_loop(0, UPW // 2, pair, 0)
    pltpu.make_async_copy(idxu.at[0], idx_v0, sem_in).wait()
    pltpu.make_async_copy(buf_v0, out.at[0, pl.ds(96, 96), pl.ds(0, 128)],
                          sem_out).wait()
    pltpu.make_async_copy(buf_v1, out.at[0, pl.ds(96, 96), pl.ds(0, 128)],
                          sem_out).wait()


@functools.partial(
    pl.kernel,
    mesh=plsc.VectorSubcoreMesh(core_axis_name="c", subcore_axis_name="s"),
    out_type=jax.ShapeDtypeStruct((NW, SPW, H), jnp.float32),
    scratch_types=[
        pltpu.VMEM((1, SPW), jnp.int32),
        pltpu.VMEM((SPW, H), jnp.float32),
        pltpu.SemaphoreType.DMA,
    ],
    compiler_params=pltpu.CompilerParams(use_tc_tiling_on_sc=False),
)
def _sc_static_gather(tstat, idxs, out, sidx_v, srows_v, sem):
    wid = lax.axis_index("s") * NC + lax.axis_index("c")
    pltpu.sync_copy(idxs.at[wid], sidx_v)
    pltpu.async_copy(tstat.at[sidx_v.at[0]], srows_v, sem).wait()
    pltpu.sync_copy(srows_v, out.at[wid])


def _tc_body(kpre_ref, knr_ref, obs_ref, mk_ref, mo_ref, known_ref, obs_out_ref):
    del kpre_ref
    known_ref[0] = jax.lax.dot(mk_ref[...], knr_ref[0], precision="highest",
                               preferred_element_type=jnp.float32)
    obs_out_ref[0] = jax.lax.dot(mo_ref[...], obs_ref[0], precision="highest",
                                 preferred_element_type=jnp.float32)


def _tc_combine(kpre, knr_aug, obs_aug, mk, mo):
    return pl.pallas_call(
        _tc_body,
        grid=(T,),
        in_specs=[
            pl.BlockSpec(memory_space=pl.ANY),
            pl.BlockSpec((1, 4, B), lambda t: (t, 0, 0)),
            pl.BlockSpec((1, 4, B), lambda t: (t, 0, 0)),
            pl.BlockSpec((3 * H, 4), lambda t: (0, 0)),
            pl.BlockSpec((3 * H, 4), lambda t: (0, 0)),
        ],
        out_specs=[
            pl.BlockSpec((1, 3 * H, B), lambda t: (t, 0, 0)),
            pl.BlockSpec((1, 3 * H, B), lambda t: (t, 0, 0)),
        ],
        out_shape=[
            jax.ShapeDtypeStruct((T, 6 * H, B), jnp.float32),
            jax.ShapeDtypeStruct((T, 3 * H, B), jnp.float32),
        ],
        input_output_aliases={0: 0},
    )(kpre, knr_aug, obs_aug, mk, mo)


def kernel(static, known_real, known_categorical, observed,
           static_emb, known_cat_emb,
           known_real_W, known_real_b, observed_W, observed_b):
    f32 = jnp.float32

    tabT = jnp.transpose(known_cat_emb, (0, 2, 1))
    tab1d = jnp.pad(tabT, ((0, 0), (0, 0), (0, VPAD - KNOWN_VOCAB))).reshape(-1)
    idxT = jnp.transpose(known_categorical, (1, 2, 0))
    idxu = (idxT.reshape(T, N_KNOWN_CAT, 8, 128)
            .transpose(0, 2, 1, 3).reshape(UNITS, N_KNOWN_CAT, 128))
    kpre = _sc_cat_gather(tab1d, idxu)

    tstat = static_emb.reshape(N_STATIC * STATIC_VOCAB, H)
    idx_stat = (static + STATIC_VOCAB * jnp.arange(N_STATIC, dtype=jnp.int32)
                ).reshape(NW, 1, SPW)
    static_rows = _sc_static_gather(tstat, idx_stat)
    static_out = static_rows.reshape(B, N_STATIC, H)

    rr = jnp.arange(3 * H)
    mk = (jnp.zeros((3 * H, 4), f32)
          .at[rr, rr // H].set(known_real_W.reshape(-1))
          .at[:, 3].set(known_real_b.reshape(-1)))
    mo = (jnp.zeros((3 * H, 4), f32)
          .at[rr, rr // H].set(observed_W.reshape(-1))
          .at[:, 3].set(observed_b.reshape(-1)))
    ones_row = jnp.ones((T, 1, B), f32)
    knr_aug = jnp.concatenate([jnp.transpose(known_real, (1, 2, 0)), ones_row], axis=1)
    obs_aug = jnp.concatenate([jnp.transpose(observed, (1, 2, 0)), ones_row], axis=1)

    known_pre, obs_pre = _tc_combine(kpre, knr_aug, obs_aug, mk, mo)

    known_out = jnp.transpose(known_pre.reshape(T, 6, H, B), (3, 0, 2, 1))
    observed_out = jnp.transpose(obs_pre.reshape(T, 3, H, B), (3, 0, 2, 1))
    return static_out, known_out, observed_out


# speedup vs baseline: 6.3642x; 1.1096x over previous
import functools

import jax
import jax.numpy as jnp
from jax import lax
from jax.experimental import pallas as pl
from jax.experimental.pallas import tpu as pltpu
from jax.experimental.pallas import tpu_sc as plsc

H = 32
B = 1024
T = 200
N_STATIC = 4
STATIC_VOCAB = 100000
N_KNOWN_CAT = 3
KNOWN_VOCAB = 1000
VPAD = 1024

NC = 2
NS = 16
NW = NC * NS

UNITS = T * (B // 128)
UPW = UNITS // NW
TABF = N_KNOWN_CAT * H * VPAD

STATIC_ROWS = B * N_STATIC
SPW = STATIC_ROWS // NW


@functools.partial(
    pl.kernel,
    mesh=plsc.VectorSubcoreMesh(core_axis_name="c", subcore_axis_name="s"),
    out_type=jax.ShapeDtypeStruct((T, 6 * H, B), jnp.float32),
    scratch_types=[
        pltpu.VMEM((TABF,), jnp.float32),
        pltpu.VMEM((N_KNOWN_CAT, 128), jnp.int32),
        pltpu.VMEM((N_KNOWN_CAT, 128), jnp.int32),
        pltpu.VMEM((N_KNOWN_CAT * H, 128), jnp.float32),
        pltpu.VMEM((N_KNOWN_CAT * H, 128), jnp.float32),
        pltpu.SemaphoreType.DMA,
        pltpu.SemaphoreType.DMA,
    ],
    compiler_params=pltpu.CompilerParams(needs_layout_passes=False),
)
def _sc_cat_gather(tab1d, idxu, out, tab_v, idx_v0, idx_v1, buf_v0, buf_v1,
                   sem_in, sem_out):
    wid = lax.axis_index("s") * NC + lax.axis_index("c")
    pltpu.sync_copy(tab1d, tab_v)
    idx_bufs = (idx_v0, idx_v1)
    bufs = (buf_v0, buf_v1)

    pltpu.async_copy(idxu.at[wid * UPW], idx_v0, sem_in)

    def pair(p, carry):
        for k in range(2):
            u = 2 * p + k
            uid = wid * UPW + u
            t = uid // 8
            b0 = pl.multiple_of((uid % 8) * 128, 128)
            icur, bcur = idx_bufs[k], bufs[k]
            pltpu.make_async_copy(idxu.at[uid], icur, sem_in).wait()
            nxt = lax.min(uid + 1, UNITS - 1)
            pltpu.async_copy(idxu.at[nxt], idx_bufs[1 - k], sem_in)

            @pl.when(u >= 2)
            def _():
                pltpu.make_async_copy(
                    bcur, out.at[0, pl.ds(96, 96), pl.ds(0, 128)], sem_out
                ).wait()

            iv = [[icur[c, pl.ds(16 * bb, 16)] for bb in range(8)]
                  for c in range(N_KNOWN_CAT)]
            prev, prev_row = None, 0
            for c in range(N_KNOWN_CAT):
                for h in range(H):
                    row = tab_v.at[pl.ds((c * H + h) * VPAD, VPAD)]
                    cur = []
                    for bb in range(8):
                        cur.append(plsc.load_gather(row, [iv[c][bb]]))
                        if prev is not None:
                            bcur[prev_row, pl.ds(16 * bb, 16)] = prev[bb]
                    prev, prev_row = cur, c * H + h
            for bb in range(8):
                bcur[prev_row, pl.ds(16 * bb, 16)] = prev[bb]
            pltpu.async_copy(bcur, out.at[t, pl.ds(96, 96), pl.ds(b0, 128)],
                             sem_out)
        return carry

    lax.fori_loop(0, UPW // 2, pair, 0)
    pltpu.make_async_copy(idxu.at[0], idx_v0, sem_in).wait()
    pltpu.make_async_copy(buf_v0, out.at[0, pl.ds(96, 96), pl.ds(0, 128)],
                          sem_out).wait()
    pltpu.make_async_copy(buf_v1, out.at[0, pl.ds(96, 96), pl.ds(0, 128)],
                          sem_out).wait()


@functools.partial(
    pl.kernel,
    mesh=plsc.VectorSubcoreMesh(core_axis_name="c", subcore_axis_name="s"),
    out_type=jax.ShapeDtypeStruct((NW, SPW, H), jnp.float32),
    scratch_types=[
        pltpu.VMEM((1, SPW), jnp.int32),
        pltpu.VMEM((SPW, H), jnp.float32),
        pltpu.SemaphoreType.DMA,
    ],
    compiler_params=pltpu.CompilerParams(use_tc_tiling_on_sc=False),
)
def _sc_static_gather(tstat, idxs, out, sidx_v, srows_v, sem):
    wid = lax.axis_index("s") * NC + lax.axis_index("c")
    pltpu.sync_copy(idxs.at[wid], sidx_v)
    pltpu.async_copy(tstat.at[sidx_v.at[0]], srows_v, sem).wait()
    pltpu.sync_copy(srows_v, out.at[wid])


TCB = 4


def _tc_body(kpre_ref, knr_ref, obs_ref, mk_ref, mo_ref, known_ref, obs_out_ref):
    del kpre_ref
    for tt in range(TCB):
        known_ref[tt] = jax.lax.dot(mk_ref[...], knr_ref[tt],
                                    precision="highest",
                                    preferred_element_type=jnp.float32)
        obs_out_ref[tt] = jax.lax.dot(mo_ref[...], obs_ref[tt],
                                      precision="highest",
                                      preferred_element_type=jnp.float32)


def _tc_combine(kpre, knr_aug, obs_aug, mk, mo):
    return pl.pallas_call(
        _tc_body,
        grid=(T // TCB,),
        in_specs=[
            pl.BlockSpec(memory_space=pl.ANY),
            pl.BlockSpec((TCB, 4, B), lambda t: (t, 0, 0)),
            pl.BlockSpec((TCB, 4, B), lambda t: (t, 0, 0)),
            pl.BlockSpec((3 * H, 4), lambda t: (0, 0)),
            pl.BlockSpec((3 * H, 4), lambda t: (0, 0)),
        ],
        out_specs=[
            pl.BlockSpec((TCB, 3 * H, B), lambda t: (t, 0, 0)),
            pl.BlockSpec((TCB, 3 * H, B), lambda t: (t, 0, 0)),
        ],
        out_shape=[
            jax.ShapeDtypeStruct((T, 6 * H, B), jnp.float32),
            jax.ShapeDtypeStruct((T, 3 * H, B), jnp.float32),
        ],
        input_output_aliases={0: 0},
    )(kpre, knr_aug, obs_aug, mk, mo)


def kernel(static, known_real, known_categorical, observed,
           static_emb, known_cat_emb,
           known_real_W, known_real_b, observed_W, observed_b):
    f32 = jnp.float32

    tabT = jnp.transpose(known_cat_emb, (0, 2, 1))
    tab1d = jnp.pad(tabT, ((0, 0), (0, 0), (0, VPAD - KNOWN_VOCAB))).reshape(-1)
    idxT = jnp.transpose(known_categorical, (1, 2, 0))
    idxu = (idxT.reshape(T, N_KNOWN_CAT, 8, 128)
            .transpose(0, 2, 1, 3).reshape(UNITS, N_KNOWN_CAT, 128))
    kpre = _sc_cat_gather(tab1d, idxu)

    tstat = static_emb.reshape(N_STATIC * STATIC_VOCAB, H)
    idx_stat = (static + STATIC_VOCAB * jnp.arange(N_STATIC, dtype=jnp.int32)
                ).reshape(NW, 1, SPW)
    static_rows = _sc_static_gather(tstat, idx_stat)
    static_out = static_rows.reshape(B, N_STATIC, H)

    rr = jnp.arange(3 * H)
    mk = (jnp.zeros((3 * H, 4), f32)
          .at[rr, rr // H].set(known_real_W.reshape(-1))
          .at[:, 3].set(known_real_b.reshape(-1)))
    mo = (jnp.zeros((3 * H, 4), f32)
          .at[rr, rr // H].set(observed_W.reshape(-1))
          .at[:, 3].set(observed_b.reshape(-1)))
    ones_row = jnp.ones((T, 1, B), f32)
    knr_aug = jnp.concatenate([jnp.transpose(known_real, (1, 2, 0)), ones_row], axis=1)
    obs_aug = jnp.concatenate([jnp.transpose(observed, (1, 2, 0)), ones_row], axis=1)

    known_pre, obs_pre = _tc_combine(kpre, knr_aug, obs_aug, mk, mo)

    known_out = jnp.transpose(known_pre.reshape(T, 6, H, B), (3, 0, 2, 1))
    observed_out = jnp.transpose(obs_pre.reshape(T, 3, H, B), (3, 0, 2, 1))
    return static_out, known_out, observed_out


# speedup vs baseline: 7.1138x; 1.1178x over previous
import functools

import jax
import jax.numpy as jnp
from jax import lax
from jax.experimental import pallas as pl
from jax.experimental.pallas import tpu as pltpu
from jax.experimental.pallas import tpu_sc as plsc

H = 32
B = 1024
T = 200
N_STATIC = 4
STATIC_VOCAB = 100000
N_KNOWN_CAT = 3
KNOWN_VOCAB = 1000
VPAD = 1024

NC = 2
NS = 16
NW = NC * NS

UNITS = T * (B // 128)
UPW = UNITS // NW
TABF = N_KNOWN_CAT * H * VPAD

STATIC_ROWS = B * N_STATIC
SPW = STATIC_ROWS // NW


@functools.partial(
    pl.kernel,
    mesh=plsc.VectorSubcoreMesh(core_axis_name="c", subcore_axis_name="s"),
    out_type=jax.ShapeDtypeStruct((T, 6 * H, B), jnp.float32),
    scratch_types=[
        pltpu.VMEM((TABF,), jnp.float32),
        pltpu.VMEM((N_KNOWN_CAT, 128), jnp.int32),
        pltpu.VMEM((N_KNOWN_CAT, 128), jnp.int32),
        pltpu.VMEM((N_KNOWN_CAT * H, 128), jnp.float32),
        pltpu.VMEM((N_KNOWN_CAT * H, 128), jnp.float32),
        pltpu.SemaphoreType.DMA,
        pltpu.SemaphoreType.DMA,
    ],
    compiler_params=pltpu.CompilerParams(needs_layout_passes=False),
)
def _sc_cat_gather(tab1d, idxu, out, tab_v, idx_v0, idx_v1, buf_v0, buf_v1,
                   sem_in, sem_out):
    wid = lax.axis_index("s") * NC + lax.axis_index("c")
    pltpu.sync_copy(tab1d, tab_v)
    idx_bufs = (idx_v0, idx_v1)
    bufs = (buf_v0, buf_v1)

    pltpu.async_copy(idxu.at[wid * UPW], idx_v0, sem_in)

    def pair(p, carry):
        for k in range(2):
            u = 2 * p + k
            uid = wid * UPW + u
            t = uid // 8
            b0 = pl.multiple_of((uid % 8) * 128, 128)
            icur, bcur = idx_bufs[k], bufs[k]
            pltpu.make_async_copy(idxu.at[uid], icur, sem_in).wait()
            nxt = lax.min(uid + 1, UNITS - 1)
            pltpu.async_copy(idxu.at[nxt], idx_bufs[1 - k], sem_in)

            @pl.when(u >= 2)
            def _():
                pltpu.make_async_copy(
                    bcur, out.at[0, pl.ds(96, 96), pl.ds(0, 128)], sem_out
                ).wait()

            iv = [[icur[c, pl.ds(16 * bb, 16)] for bb in range(8)]
                  for c in range(N_KNOWN_CAT)]
            prev, prev_row = None, 0
            for c in range(N_KNOWN_CAT):
                for h in range(H):
                    row = tab_v.at[pl.ds((c * H + h) * VPAD, VPAD)]
                    cur = []
                    for bb in range(8):
                        cur.append(plsc.load_gather(row, [iv[c][bb]]))
                        if prev is not None:
                            bcur[prev_row, pl.ds(16 * bb, 16)] = prev[bb]
                    prev, prev_row = cur, c * H + h
            for bb in range(8):
                bcur[prev_row, pl.ds(16 * bb, 16)] = prev[bb]
            pltpu.async_copy(bcur, out.at[t, pl.ds(96, 96), pl.ds(b0, 128)],
                             sem_out)
        return carry

    lax.fori_loop(0, UPW // 2, pair, 0)
    pltpu.make_async_copy(idxu.at[0], idx_v0, sem_in).wait()
    pltpu.make_async_copy(buf_v0, out.at[0, pl.ds(96, 96), pl.ds(0, 128)],
                          sem_out).wait()
    pltpu.make_async_copy(buf_v1, out.at[0, pl.ds(96, 96), pl.ds(0, 128)],
                          sem_out).wait()


@functools.partial(
    pl.kernel,
    mesh=plsc.VectorSubcoreMesh(core_axis_name="c", subcore_axis_name="s"),
    out_type=jax.ShapeDtypeStruct((NW, SPW, H), jnp.float32),
    scratch_types=[
        pltpu.VMEM((1, SPW), jnp.int32),
        pltpu.VMEM((SPW, H), jnp.float32),
        pltpu.SemaphoreType.DMA,
    ],
    compiler_params=pltpu.CompilerParams(use_tc_tiling_on_sc=False),
)
def _sc_static_gather(tstat, idxs, out, sidx_v, srows_v, sem):
    wid = lax.axis_index("s") * NC + lax.axis_index("c")
    pltpu.sync_copy(idxs.at[wid], sidx_v)
    pltpu.async_copy(tstat.at[sidx_v.at[0]], srows_v, sem).wait()
    pltpu.sync_copy(srows_v, out.at[wid])


TCB = 4


def _tc_known_body(kpre_ref, knr_ref, mk_ref, known_ref):
    del kpre_ref
    for tt in range(TCB):
        known_ref[tt] = jax.lax.dot(mk_ref[...], knr_ref[tt],
                                    preferred_element_type=jnp.float32)


def _tc_obs_body(obs_ref, mo_ref, obs_out_ref):
    for tt in range(TCB):
        obs_out_ref[tt] = jax.lax.dot(mo_ref[...], obs_ref[tt],
                                      preferred_element_type=jnp.float32)


def _tc_known(kpre, knr_aug, mk):
    return pl.pallas_call(
        _tc_known_body,
        grid=(T // TCB,),
        in_specs=[
            pl.BlockSpec(memory_space=pl.ANY),
            pl.BlockSpec((TCB, 4, B), lambda t: (t, 0, 0)),
            pl.BlockSpec((3 * H, 4), lambda t: (0, 0)),
        ],
        out_specs=pl.BlockSpec((TCB, 3 * H, B), lambda t: (t, 0, 0)),
        out_shape=jax.ShapeDtypeStruct((T, 6 * H, B), jnp.float32),
        input_output_aliases={0: 0},
    )(kpre, knr_aug, mk)


def _tc_obs(obs_aug, mo):
    return pl.pallas_call(
        _tc_obs_body,
        grid=(T // TCB,),
        in_specs=[
            pl.BlockSpec((TCB, 4, B), lambda t: (t, 0, 0)),
            pl.BlockSpec((3 * H, 4), lambda t: (0, 0)),
        ],
        out_specs=pl.BlockSpec((TCB, 3 * H, B), lambda t: (t, 0, 0)),
        out_shape=jax.ShapeDtypeStruct((T, 3 * H, B), jnp.float32),
    )(obs_aug, mo)


def kernel(static, known_real, known_categorical, observed,
           static_emb, known_cat_emb,
           known_real_W, known_real_b, observed_W, observed_b):
    f32 = jnp.float32

    tabT = jnp.transpose(known_cat_emb, (0, 2, 1))
    tab1d = jnp.pad(tabT, ((0, 0), (0, 0), (0, VPAD - KNOWN_VOCAB))).reshape(-1)
    idxT = jnp.transpose(known_categorical, (1, 2, 0))
    idxu = (idxT.reshape(T, N_KNOWN_CAT, 8, 128)
            .transpose(0, 2, 1, 3).reshape(UNITS, N_KNOWN_CAT, 128))
    kpre = _sc_cat_gather(tab1d, idxu)

    tstat = static_emb.reshape(N_STATIC * STATIC_VOCAB, H)
    idx_stat = (static + STATIC_VOCAB * jnp.arange(N_STATIC, dtype=jnp.int32)
                ).reshape(NW, 1, SPW)
    static_rows = _sc_static_gather(tstat, idx_stat)
    static_out = static_rows.reshape(B, N_STATIC, H)

    rr = jnp.arange(3 * H)
    mk = (jnp.zeros((3 * H, 4), f32)
          .at[rr, rr // H].set(known_real_W.reshape(-1))
          .at[:, 3].set(known_real_b.reshape(-1)))
    mo = (jnp.zeros((3 * H, 4), f32)
          .at[rr, rr // H].set(observed_W.reshape(-1))
          .at[:, 3].set(observed_b.reshape(-1)))
    ones_row = jnp.ones((T, 1, B), f32)
    knr_aug = jnp.concatenate([jnp.transpose(known_real, (1, 2, 0)), ones_row], axis=1)
    obs_aug = jnp.concatenate([jnp.transpose(observed, (1, 2, 0)), ones_row], axis=1)

    obs_pre = _tc_obs(obs_aug, mo)
    known_pre = _tc_known(kpre, knr_aug, mk)

    known_out = jnp.transpose(known_pre.reshape(T, 6, H, B), (3, 0, 2, 1))
    observed_out = jnp.transpose(obs_pre.reshape(T, 3, H, B), (3, 0, 2, 1))
    return static_out, known_out, observed_out


# speedup vs baseline: 7.6763x; 1.0791x over previous
import functools

import jax
import jax.numpy as jnp
from jax import lax
from jax.experimental import pallas as pl
from jax.experimental.pallas import tpu as pltpu
from jax.experimental.pallas import tpu_sc as plsc

H = 32
B = 1024
T = 200
N_STATIC = 4
STATIC_VOCAB = 100000
N_KNOWN_CAT = 3
KNOWN_VOCAB = 1000
VPAD = 1024

NC = 2
NS = 16
NW = NC * NS

UNITS = T * (B // 128)
UPW = UNITS // NW
TABF = N_KNOWN_CAT * H * VPAD

STATIC_ROWS = B * N_STATIC
SPW = STATIC_ROWS // NW


@functools.partial(
    pl.kernel,
    mesh=plsc.VectorSubcoreMesh(core_axis_name="c", subcore_axis_name="s"),
    out_type=jax.ShapeDtypeStruct((T, 6 * H, B), jnp.float32),
    scratch_types=[
        pltpu.VMEM((TABF,), jnp.float32),
        pltpu.VMEM((N_KNOWN_CAT, 128), jnp.int32),
        pltpu.VMEM((N_KNOWN_CAT, 128), jnp.int32),
        pltpu.VMEM((N_KNOWN_CAT * H, 128), jnp.float32),
        pltpu.VMEM((N_KNOWN_CAT * H, 128), jnp.float32),
        pltpu.SemaphoreType.DMA,
        pltpu.SemaphoreType.DMA,
    ],
    compiler_params=pltpu.CompilerParams(needs_layout_passes=False),
)
def _sc_cat_gather(tab1d, idxu, out, tab_v, idx_v0, idx_v1, buf_v0, buf_v1,
                   sem_in, sem_out):
    wid = lax.axis_index("s") * NC + lax.axis_index("c")
    pltpu.sync_copy(tab1d, tab_v)
    idx_bufs = (idx_v0, idx_v1)
    bufs = (buf_v0, buf_v1)

    pltpu.async_copy(idxu.at[wid * UPW], idx_v0, sem_in)

    def pair(p, carry):
        for k in range(2):
            u = 2 * p + k
            uid = wid * UPW + u
            t = uid // 8
            b0 = pl.multiple_of((uid % 8) * 128, 128)
            icur, bcur = idx_bufs[k], bufs[k]
            pltpu.make_async_copy(idxu.at[uid], icur, sem_in).wait()
            nxt = lax.min(uid + 1, UNITS - 1)
            pltpu.async_copy(idxu.at[nxt], idx_bufs[1 - k], sem_in)

            @pl.when(u >= 2)
            def _():
                pltpu.make_async_copy(
                    bcur, out.at[0, pl.ds(96, 96), pl.ds(0, 128)], sem_out
                ).wait()

            iv = [[icur[c, pl.ds(16 * bb, 16)] for bb in range(8)]
                  for c in range(N_KNOWN_CAT)]
            prev, prev_row = None, 0
            for c in range(N_KNOWN_CAT):
                for h in range(H):
                    row = tab_v.at[pl.ds((c * H + h) * VPAD, VPAD)]
                    cur = []
                    for bb in range(8):
                        cur.append(plsc.load_gather(row, [iv[c][bb]]))
                        if prev is not None:
                            bcur[prev_row, pl.ds(16 * bb, 16)] = prev[bb]
                    prev, prev_row = cur, c * H + h
            for bb in range(8):
                bcur[prev_row, pl.ds(16 * bb, 16)] = prev[bb]
            pltpu.async_copy(bcur, out.at[t, pl.ds(96, 96), pl.ds(b0, 128)],
                             sem_out)
        return carry

    lax.fori_loop(0, UPW // 2, pair, 0)
    pltpu.make_async_copy(idxu.at[0], idx_v0, sem_in).wait()
    pltpu.make_async_copy(buf_v0, out.at[0, pl.ds(96, 96), pl.ds(0, 128)],
                          sem_out).wait()
    pltpu.make_async_copy(buf_v1, out.at[0, pl.ds(96, 96), pl.ds(0, 128)],
                          sem_out).wait()


@functools.partial(
    pl.kernel,
    mesh=plsc.VectorSubcoreMesh(core_axis_name="c", subcore_axis_name="s"),
    out_type=jax.ShapeDtypeStruct((NW, SPW, H), jnp.float32),
    scratch_types=[
        pltpu.VMEM((1, SPW), jnp.int32),
        pltpu.VMEM((SPW, H), jnp.float32),
        pltpu.SemaphoreType.DMA,
    ],
    compiler_params=pltpu.CompilerParams(use_tc_tiling_on_sc=False),
)
def _sc_static_gather(tstat, idxs, out, sidx_v, srows_v, sem):
    wid = lax.axis_index("s") * NC + lax.axis_index("c")
    pltpu.sync_copy(idxs.at[wid], sidx_v)
    pltpu.async_copy(tstat.at[sidx_v.at[0]], srows_v, sem).wait()
    pltpu.sync_copy(srows_v, out.at[wid])


TCB = 8


def _tc_known_body(kpre_ref, knr_ref, mk_ref, known_ref):
    del kpre_ref
    for tt in range(TCB):
        known_ref[tt] = jax.lax.dot(mk_ref[...], knr_ref[tt],
                                    preferred_element_type=jnp.float32)


def _tc_obs_body(obs_ref, mo_ref, obs_out_ref):
    for tt in range(TCB):
        obs_out_ref[tt] = jax.lax.dot(mo_ref[...], obs_ref[tt],
                                      preferred_element_type=jnp.float32)


def _tc_known(kpre, knr_aug, mk):
    return pl.pallas_call(
        _tc_known_body,
        grid=(T // TCB,),
        in_specs=[
            pl.BlockSpec(memory_space=pl.ANY),
            pl.BlockSpec((TCB, 4, B), lambda t: (t, 0, 0)),
            pl.BlockSpec((3 * H, 4), lambda t: (0, 0)),
        ],
        out_specs=pl.BlockSpec((TCB, 3 * H, B), lambda t: (t, 0, 0)),
        out_shape=jax.ShapeDtypeStruct((T, 6 * H, B), jnp.float32),
        input_output_aliases={0: 0},
    )(kpre, knr_aug, mk)


def _tc_obs(obs_aug, mo):
    return pl.pallas_call(
        _tc_obs_body,
        grid=(T // TCB,),
        in_specs=[
            pl.BlockSpec((TCB, 4, B), lambda t: (t, 0, 0)),
            pl.BlockSpec((3 * H, 4), lambda t: (0, 0)),
        ],
        out_specs=pl.BlockSpec((TCB, 3 * H, B), lambda t: (t, 0, 0)),
        out_shape=jax.ShapeDtypeStruct((T, 3 * H, B), jnp.float32),
    )(obs_aug, mo)


def kernel(static, known_real, known_categorical, observed,
           static_emb, known_cat_emb,
           known_real_W, known_real_b, observed_W, observed_b):
    f32 = jnp.float32

    tabT = jnp.transpose(known_cat_emb, (0, 2, 1))
    tab1d = jnp.pad(tabT, ((0, 0), (0, 0), (0, VPAD - KNOWN_VOCAB))).reshape(-1)
    idxT = jnp.transpose(known_categorical, (1, 2, 0))
    idxu = (idxT.reshape(T, N_KNOWN_CAT, 8, 128)
            .transpose(0, 2, 1, 3).reshape(UNITS, N_KNOWN_CAT, 128))
    kpre = _sc_cat_gather(tab1d, idxu)

    rr = jnp.arange(3 * H)
    mk = (jnp.zeros((3 * H, 4), f32)
          .at[rr, rr // H].set(known_real_W.reshape(-1))
          .at[:, 3].set(known_real_b.reshape(-1)))
    mo = (jnp.zeros((3 * H, 4), f32)
          .at[rr, rr // H].set(observed_W.reshape(-1))
          .at[:, 3].set(observed_b.reshape(-1)))
    ones_row = jnp.ones((T, 1, B), f32)
    knr_aug = jnp.concatenate([jnp.transpose(known_real, (1, 2, 0)), ones_row], axis=1)
    obs_aug = jnp.concatenate([jnp.transpose(observed, (1, 2, 0)), ones_row], axis=1)

    obs_pre = _tc_obs(obs_aug, mo)
    known_pre = _tc_known(kpre, knr_aug, mk)

    tstat = static_emb.reshape(N_STATIC * STATIC_VOCAB, H)
    idx_stat = (static + STATIC_VOCAB * jnp.arange(N_STATIC, dtype=jnp.int32)
                ).reshape(NW, 1, SPW)
    static_rows = _sc_static_gather(tstat, idx_stat)
    static_out = static_rows.reshape(B, N_STATIC, H)

    known_out = jnp.transpose(known_pre.reshape(T, 6, H, B), (3, 0, 2, 1))
    observed_out = jnp.transpose(obs_pre.reshape(T, 3, H, B), (3, 0, 2, 1))
    return static_out, known_out, observed_out


# speedup vs baseline: 7.8237x; 1.0192x over previous
import functools

import jax
import jax.numpy as jnp
from jax import lax
from jax.experimental import pallas as pl
from jax.experimental.pallas import tpu as pltpu
from jax.experimental.pallas import tpu_sc as plsc

H = 32
B = 1024
T = 200
N_STATIC = 4
STATIC_VOCAB = 100000
N_KNOWN_CAT = 3
KNOWN_VOCAB = 1000
VPAD = 1024

NC = 2
NS = 16
NW = NC * NS

UNITS = T * (B // 128)
UPW = UNITS // NW
TABF = N_KNOWN_CAT * H * VPAD

STATIC_ROWS = B * N_STATIC
SPW = STATIC_ROWS // NW


@functools.partial(
    pl.kernel,
    mesh=plsc.VectorSubcoreMesh(core_axis_name="c", subcore_axis_name="s"),
    out_type=jax.ShapeDtypeStruct((T, 6 * H, B), jnp.float32),
    scratch_types=[
        pltpu.VMEM((TABF,), jnp.float32),
        pltpu.VMEM((N_KNOWN_CAT, 128), jnp.int32),
        pltpu.VMEM((N_KNOWN_CAT, 128), jnp.int32),
        pltpu.VMEM((N_KNOWN_CAT * H, 128), jnp.float32),
        pltpu.VMEM((N_KNOWN_CAT * H, 128), jnp.float32),
        pltpu.SemaphoreType.DMA,
        pltpu.SemaphoreType.DMA,
    ],
    compiler_params=pltpu.CompilerParams(needs_layout_passes=False),
)
def _sc_cat_gather(tab1d, idxu, out, tab_v, idx_v0, idx_v1, buf_v0, buf_v1,
                   sem_in, sem_out):
    wid = lax.axis_index("s") * NC + lax.axis_index("c")
    pltpu.sync_copy(tab1d, tab_v)
    idx_bufs = (idx_v0, idx_v1)
    bufs = (buf_v0, buf_v1)

    pltpu.async_copy(idxu.at[wid * UPW], idx_v0, sem_in)

    def pair(p, carry):
        for k in range(2):
            u = 2 * p + k
            uid = wid * UPW + u
            t = uid // 8
            b0 = pl.multiple_of((uid % 8) * 128, 128)
            icur, bcur = idx_bufs[k], bufs[k]
            pltpu.make_async_copy(idxu.at[uid], icur, sem_in).wait()
            nxt = lax.min(uid + 1, UNITS - 1)
            pltpu.async_copy(idxu.at[nxt], idx_bufs[1 - k], sem_in)

            @pl.when(u >= 2)
            def _():
                pltpu.make_async_copy(
                    bcur, out.at[0, pl.ds(96, 96), pl.ds(0, 128)], sem_out
                ).wait()

            iv = [[icur[c, pl.ds(16 * bb, 16)] for bb in range(8)]
                  for c in range(N_KNOWN_CAT)]
            prev, prev_row = None, 0
            for c in range(N_KNOWN_CAT):
                for h in range(H):
                    row = tab_v.at[pl.ds((c * H + h) * VPAD, VPAD)]
                    cur = []
                    for bb in range(8):
                        cur.append(plsc.load_gather(row, [iv[c][bb]]))
                        if prev is not None:
                            bcur[prev_row, pl.ds(16 * bb, 16)] = prev[bb]
                    prev, prev_row = cur, c * H + h
            for bb in range(8):
                bcur[prev_row, pl.ds(16 * bb, 16)] = prev[bb]
            pltpu.async_copy(bcur, out.at[t, pl.ds(96, 96), pl.ds(b0, 128)],
                             sem_out)
        return carry

    lax.fori_loop(0, UPW // 2, pair, 0)
    pltpu.make_async_copy(idxu.at[0], idx_v0, sem_in).wait()
    pltpu.make_async_copy(buf_v0, out.at[0, pl.ds(96, 96), pl.ds(0, 128)],
                          sem_out).wait()
    pltpu.make_async_copy(buf_v1, out.at[0, pl.ds(96, 96), pl.ds(0, 128)],
                          sem_out).wait()


@functools.partial(
    pl.kernel,
    mesh=plsc.VectorSubcoreMesh(core_axis_name="c", subcore_axis_name="s"),
    out_type=jax.ShapeDtypeStruct((NW, SPW, H), jnp.float32),
    scratch_types=[
        pltpu.VMEM((1, SPW), jnp.int32),
        pltpu.VMEM((SPW, H), jnp.float32),
        pltpu.SemaphoreType.DMA,
    ],
    compiler_params=pltpu.CompilerParams(use_tc_tiling_on_sc=False),
)
def _sc_static_gather(tstat, idxs, out, sidx_v, srows_v, sem):
    wid = lax.axis_index("s") * NC + lax.axis_index("c")
    pltpu.sync_copy(idxs.at[wid], sidx_v)
    pltpu.async_copy(tstat.at[sidx_v.at[0]], srows_v, sem).wait()
    pltpu.sync_copy(srows_v, out.at[wid])


TCB = 10


def _tc_known_body(kpre_ref, knr_ref, mk_ref, known_ref):
    del kpre_ref
    for tt in range(TCB):
        known_ref[tt] = jax.lax.dot(mk_ref[...], knr_ref[tt],
                                    preferred_element_type=jnp.float32)


def _tc_obs_body(obs_ref, mo_ref, obs_out_ref):
    for tt in range(TCB):
        obs_out_ref[tt] = jax.lax.dot(mo_ref[...], obs_ref[tt],
                                      preferred_element_type=jnp.float32)


def _tc_known(kpre, knr_aug, mk):
    return pl.pallas_call(
        _tc_known_body,
        grid=(T // TCB,),
        in_specs=[
            pl.BlockSpec(memory_space=pl.ANY),
            pl.BlockSpec((TCB, 4, B), lambda t: (t, 0, 0)),
            pl.BlockSpec((3 * H, 4), lambda t: (0, 0)),
        ],
        out_specs=pl.BlockSpec((TCB, 3 * H, B), lambda t: (t, 0, 0)),
        out_shape=jax.ShapeDtypeStruct((T, 6 * H, B), jnp.float32),
        input_output_aliases={0: 0},
    )(kpre, knr_aug, mk)


def _tc_obs(obs_aug, mo):
    return pl.pallas_call(
        _tc_obs_body,
        grid=(T // TCB,),
        in_specs=[
            pl.BlockSpec((TCB, 4, B), lambda t: (t, 0, 0)),
            pl.BlockSpec((3 * H, 4), lambda t: (0, 0)),
        ],
        out_specs=pl.BlockSpec((TCB, 3 * H, B), lambda t: (t, 0, 0)),
        out_shape=jax.ShapeDtypeStruct((T, 3 * H, B), jnp.float32),
    )(obs_aug, mo)


def kernel(static, known_real, known_categorical, observed,
           static_emb, known_cat_emb,
           known_real_W, known_real_b, observed_W, observed_b):
    f32 = jnp.float32

    tabT = jnp.transpose(known_cat_emb, (0, 2, 1))
    tab1d = jnp.pad(tabT, ((0, 0), (0, 0), (0, VPAD - KNOWN_VOCAB))).reshape(-1)
    idxT = jnp.transpose(known_categorical, (1, 2, 0))
    idxu = (idxT.reshape(T, N_KNOWN_CAT, 8, 128)
            .transpose(0, 2, 1, 3).reshape(UNITS, N_KNOWN_CAT, 128))
    kpre = _sc_cat_gather(tab1d, idxu)

    rr = jnp.arange(3 * H)
    mk = (jnp.zeros((3 * H, 4), f32)
          .at[rr, rr // H].set(known_real_W.reshape(-1))
          .at[:, 3].set(known_real_b.reshape(-1)))
    mo = (jnp.zeros((3 * H, 4), f32)
          .at[rr, rr // H].set(observed_W.reshape(-1))
          .at[:, 3].set(observed_b.reshape(-1)))
    ones_row = jnp.ones((T, 1, B), f32)
    knr_aug = jnp.concatenate([jnp.transpose(known_real, (1, 2, 0)), ones_row], axis=1)
    obs_aug = jnp.concatenate([jnp.transpose(observed, (1, 2, 0)), ones_row], axis=1)

    obs_pre = _tc_obs(obs_aug, mo)
    known_pre = _tc_known(kpre, knr_aug, mk)

    tstat = static_emb.reshape(N_STATIC * STATIC_VOCAB, H)
    idx_stat = (static + STATIC_VOCAB * jnp.arange(N_STATIC, dtype=jnp.int32)
                ).reshape(NW, 1, SPW)
    static_rows = _sc_static_gather(tstat, idx_stat)
    static_out = static_rows.reshape(B, N_STATIC, H)

    known_out = jnp.transpose(known_pre.reshape(T, 6, H, B), (3, 0, 2, 1))
    observed_out = jnp.transpose(obs_pre.reshape(T, 3, H, B), (3, 0, 2, 1))
    return static_out, known_out, observed_out


# speedup vs baseline: 7.9977x; 1.0222x over previous
import functools

import jax
import jax.numpy as jnp
from jax import lax
from jax.experimental import pallas as pl
from jax.experimental.pallas import tpu as pltpu
from jax.experimental.pallas import tpu_sc as plsc

H = 32
B = 1024
T = 200
N_STATIC = 4
STATIC_VOCAB = 100000
N_KNOWN_CAT = 3
KNOWN_VOCAB = 1000
VPAD = 1024

NC = 2
NS = 16
NW = NC * NS

UNITS = T * (B // 128)
UPW = UNITS // NW
TABF = N_KNOWN_CAT * H * VPAD

STATIC_ROWS = B * N_STATIC
SPW = STATIC_ROWS // NW


@functools.partial(
    pl.kernel,
    mesh=plsc.VectorSubcoreMesh(core_axis_name="c", subcore_axis_name="s"),
    out_type=jax.ShapeDtypeStruct((T, 6 * H, B), jnp.float32),
    scratch_types=[
        pltpu.VMEM((TABF,), jnp.float32),
        pltpu.VMEM((N_KNOWN_CAT, 128), jnp.int32),
        pltpu.VMEM((N_KNOWN_CAT, 128), jnp.int32),
        pltpu.VMEM((N_KNOWN_CAT * H, 128), jnp.float32),
        pltpu.VMEM((N_KNOWN_CAT * H, 128), jnp.float32),
        pltpu.SemaphoreType.DMA,
        pltpu.SemaphoreType.DMA,
    ],
    compiler_params=pltpu.CompilerParams(needs_layout_passes=False),
)
def _sc_cat_gather(tab1d, idxu, out, tab_v, idx_v0, idx_v1, buf_v0, buf_v1,
                   sem_in, sem_out):
    wid = lax.axis_index("s") * NC + lax.axis_index("c")
    pltpu.sync_copy(tab1d, tab_v)
    idx_bufs = (idx_v0, idx_v1)
    bufs = (buf_v0, buf_v1)

    pltpu.async_copy(idxu.at[wid * UPW], idx_v0, sem_in)

    def pair(p, carry):
        for k in range(2):
            u = 2 * p + k
            uid = wid * UPW + u
            t = uid // 8
            b0 = pl.multiple_of((uid % 8) * 128, 128)
            icur, bcur = idx_bufs[k], bufs[k]
            pltpu.make_async_copy(idxu.at[uid], icur, sem_in).wait()
            nxt = lax.min(uid + 1, UNITS - 1)
            pltpu.async_copy(idxu.at[nxt], idx_bufs[1 - k], sem_in)

            @pl.when(u >= 2)
            def _():
                pltpu.make_async_copy(
                    bcur, out.at[0, pl.ds(96, 96), pl.ds(0, 128)], sem_out
                ).wait()

            iv = [[icur[c, pl.ds(16 * bb, 16)] for bb in range(8)]
                  for c in range(N_KNOWN_CAT)]
            prev, prev_row = None, 0
            for c in range(N_KNOWN_CAT):
                for h in range(H):
                    row = tab_v.at[pl.ds((c * H + h) * VPAD, VPAD)]
                    cur = []
                    for bb in range(8):
                        cur.append(plsc.load_gather(row, [iv[c][bb]]))
                        if prev is not None:
                            bcur[prev_row, pl.ds(16 * bb, 16)] = prev[bb]
                    prev, prev_row = cur, c * H + h
            for bb in range(8):
                bcur[prev_row, pl.ds(16 * bb, 16)] = prev[bb]
            pltpu.async_copy(bcur, out.at[t, pl.ds(96, 96), pl.ds(b0, 128)],
                             sem_out)
        return carry

    lax.fori_loop(0, UPW // 2, pair, 0)
    pltpu.make_async_copy(idxu.at[0], idx_v0, sem_in).wait()
    pltpu.make_async_copy(buf_v0, out.at[0, pl.ds(96, 96), pl.ds(0, 128)],
                          sem_out).wait()
    pltpu.make_async_copy(buf_v1, out.at[0, pl.ds(96, 96), pl.ds(0, 128)],
                          sem_out).wait()


@functools.partial(
    pl.kernel,
    mesh=plsc.VectorSubcoreMesh(core_axis_name="c", subcore_axis_name="s"),
    out_type=jax.ShapeDtypeStruct((NW, SPW, H), jnp.float32),
    scratch_types=[
        pltpu.VMEM((1, SPW), jnp.int32),
        pltpu.VMEM((SPW, H), jnp.float32),
        pltpu.SemaphoreType.DMA,
    ],
    compiler_params=pltpu.CompilerParams(use_tc_tiling_on_sc=False),
)
def _sc_static_gather(tstat, idxs, out, sidx_v, srows_v, sem):
    wid = lax.axis_index("s") * NC + lax.axis_index("c")
    pltpu.sync_copy(idxs.at[wid], sidx_v)
    pltpu.async_copy(tstat.at[sidx_v.at[0]], srows_v, sem).wait()
    pltpu.sync_copy(srows_v, out.at[wid])


TCB = 20


def _tc_known_body(kpre_ref, knr_ref, mk_ref, known_ref):
    del kpre_ref
    for tt in range(TCB):
        known_ref[tt] = jax.lax.dot(mk_ref[...], knr_ref[tt],
                                    preferred_element_type=jnp.float32)


def _tc_obs_body(obs_ref, mo_ref, obs_out_ref):
    for tt in range(TCB):
        obs_out_ref[tt] = jax.lax.dot(mo_ref[...], obs_ref[tt],
                                      preferred_element_type=jnp.float32)


def _tc_known(kpre, knr_aug, mk):
    return pl.pallas_call(
        _tc_known_body,
        grid=(T // TCB,),
        in_specs=[
            pl.BlockSpec(memory_space=pl.ANY),
            pl.BlockSpec((TCB, 4, B), lambda t: (t, 0, 0)),
            pl.BlockSpec((3 * H, 4), lambda t: (0, 0)),
        ],
        out_specs=pl.BlockSpec((TCB, 3 * H, B), lambda t: (t, 0, 0)),
        out_shape=jax.ShapeDtypeStruct((T, 6 * H, B), jnp.float32),
        input_output_aliases={0: 0},
    )(kpre, knr_aug, mk)


def _tc_obs(obs_aug, mo):
    return pl.pallas_call(
        _tc_obs_body,
        grid=(T // TCB,),
        in_specs=[
            pl.BlockSpec((TCB, 4, B), lambda t: (t, 0, 0)),
            pl.BlockSpec((3 * H, 4), lambda t: (0, 0)),
        ],
        out_specs=pl.BlockSpec((TCB, 3 * H, B), lambda t: (t, 0, 0)),
        out_shape=jax.ShapeDtypeStruct((T, 3 * H, B), jnp.float32),
    )(obs_aug, mo)


def kernel(static, known_real, known_categorical, observed,
           static_emb, known_cat_emb,
           known_real_W, known_real_b, observed_W, observed_b):
    f32 = jnp.float32

    tabT = jnp.transpose(known_cat_emb, (0, 2, 1))
    tab1d = jnp.pad(tabT, ((0, 0), (0, 0), (0, VPAD - KNOWN_VOCAB))).reshape(-1)
    idxT = jnp.transpose(known_categorical, (1, 2, 0))
    idxu = (idxT.reshape(T, N_KNOWN_CAT, 8, 128)
            .transpose(0, 2, 1, 3).reshape(UNITS, N_KNOWN_CAT, 128))
    kpre = _sc_cat_gather(tab1d, idxu)

    rr = jnp.arange(3 * H)
    mk = (jnp.zeros((3 * H, 4), f32)
          .at[rr, rr // H].set(known_real_W.reshape(-1))
          .at[:, 3].set(known_real_b.reshape(-1)))
    mo = (jnp.zeros((3 * H, 4), f32)
          .at[rr, rr // H].set(observed_W.reshape(-1))
          .at[:, 3].set(observed_b.reshape(-1)))
    ones_row = jnp.ones((T, 1, B), f32)
    knr_aug = jnp.concatenate([jnp.transpose(known_real, (1, 2, 0)), ones_row], axis=1)
    obs_aug = jnp.concatenate([jnp.transpose(observed, (1, 2, 0)), ones_row], axis=1)

    obs_pre = _tc_obs(obs_aug, mo)
    known_pre = _tc_known(kpre, knr_aug, mk)

    tstat = static_emb.reshape(N_STATIC * STATIC_VOCAB, H)
    idx_stat = (static + STATIC_VOCAB * jnp.arange(N_STATIC, dtype=jnp.int32)
                ).reshape(NW, 1, SPW)
    static_rows = _sc_static_gather(tstat, idx_stat)
    static_out = static_rows.reshape(B, N_STATIC, H)

    known_out = jnp.transpose(known_pre.reshape(T, 6, H, B), (3, 0, 2, 1))
    observed_out = jnp.transpose(obs_pre.reshape(T, 3, H, B), (3, 0, 2, 1))
    return static_out, known_out, observed_out
